# Initial kernel scaffold; baseline (speedup 1.0000x reference)
import jax
import jax.numpy as jnp
from jax.experimental import pallas as pl


def kernel(token_ids, W):
    raise NotImplementedError("write your pallas kernel here")


# speedup vs baseline: 4.8094x; 4.8094x over previous
import functools

import jax
import jax.numpy as jnp
from jax import lax
from jax.experimental import pallas as pl
from jax.experimental.pallas import tpu as pltpu
from jax.experimental.pallas import tpu_sc as plsc


@functools.lru_cache(maxsize=None)
def _make_gather(V, D, B, chunk):
    info = plsc.get_sparse_core_info()
    NC, NS = info.num_cores, info.num_subcores
    NW = NC * NS
    assert B % NW == 0
    b_per_w = B // NW
    assert b_per_w % chunk == 0
    n_chunks = b_per_w // chunk
    mesh = plsc.VectorSubcoreMesh(core_axis_name="c", subcore_axis_name="s")

    @functools.partial(
        pl.kernel,
        mesh=mesh,
        compiler_params=pltpu.CompilerParams(use_tc_tiling_on_sc=False),
        out_type=jax.ShapeDtypeStruct((B, D), jnp.float32),
        scratch_types=[
            pltpu.VMEM((chunk,), jnp.int32),
            pltpu.VMEM((chunk, D), jnp.float32),
            pltpu.SemaphoreType.DMA,
        ],
    )
    def gather_k(table_hbm, idx_hbm, out_hbm, idx_v, rows_v, sem):
        wid = lax.axis_index("s") * NC + lax.axis_index("c")
        base = wid * b_per_w

        def body(i, carry):
            off = base + i * chunk
            pltpu.sync_copy(idx_hbm.at[pl.ds(off, chunk)], idx_v)
            pltpu.async_copy(table_hbm.at[idx_v], rows_v, sem).wait()
            pltpu.sync_copy(rows_v, out_hbm.at[pl.ds(off, chunk)])
            return carry

        lax.fori_loop(0, n_chunks, body, 0)

    return gather_k


def kernel(token_ids, W):
    S, T = token_ids.shape
    V, D = W.shape
    B = S * T
    idx = token_ids.reshape(B).astype(jnp.int32)
    out = _make_gather(V, D, B, 1024)(W, idx)
    return out.reshape(S, T, D)


# speedup vs baseline: 5.0489x; 1.0498x over previous
import functools

import jax
import jax.numpy as jnp
from jax import lax
from jax.experimental import pallas as pl
from jax.experimental.pallas import tpu as pltpu
from jax.experimental.pallas import tpu_sc as plsc


@functools.lru_cache(maxsize=None)
def _make_gather(V, D, B, chunk, nbuf):
    info = plsc.get_sparse_core_info()
    NC, NS = info.num_cores, info.num_subcores
    NW = NC * NS
    assert B % NW == 0
    b_per_w = B // NW
    assert b_per_w % chunk == 0
    n_chunks = b_per_w // chunk
    assert n_chunks % nbuf == 0
    n_groups = n_chunks // nbuf
    mesh = plsc.VectorSubcoreMesh(core_axis_name="c", subcore_axis_name="s")

    @functools.partial(
        pl.kernel,
        mesh=mesh,
        compiler_params=pltpu.CompilerParams(use_tc_tiling_on_sc=False),
        out_type=jax.ShapeDtypeStruct((B, D), jnp.float32),
        scratch_types=[
            pltpu.VMEM((nbuf, chunk), jnp.int32),
            pltpu.VMEM((nbuf, chunk, D), jnp.float32),
        ] + [pltpu.SemaphoreType.DMA] * (3 * nbuf),
    )
    def gather_k(table_hbm, idx_hbm, out_hbm, idx_v, rows_v, *sems):
        isem = sems[0:nbuf]
        gsem = sems[nbuf:2 * nbuf]
        osem = sems[2 * nbuf:3 * nbuf]
        wid = lax.axis_index("s") * NC + lax.axis_index("c")
        base = wid * b_per_w

        def idx_dma(i, b):
            off = base + (i % n_chunks) * chunk
            return pltpu.make_async_copy(
                idx_hbm.at[pl.ds(off, chunk)], idx_v.at[b], isem[b])

        def gat_dma(b):
            return pltpu.make_async_copy(
                table_hbm.at[idx_v.at[b]], rows_v.at[b], gsem[b])

        def out_dma(i, b):
            off = base + (i % n_chunks) * chunk
            return pltpu.make_async_copy(
                rows_v.at[b], out_hbm.at[pl.ds(off, chunk)], osem[b])

        for b in range(nbuf):
            idx_dma(b, b).start()
        for b in range(nbuf):
            idx_dma(b, b).wait()
            gat_dma(b).start()

        def body(g, carry):
            for b in range(nbuf):
                i = g * nbuf + b
                gat_dma(b).wait()
                out_dma(i, b).start()
                idx_dma(i + nbuf, b).start()
            for b in range(nbuf):
                i = (g + 1) * nbuf + b
                out_dma(i, b).wait()
                idx_dma(i, b).wait()
                gat_dma(b).start()
            return carry

        lax.fori_loop(0, n_groups - 1, body, 0)

        for b in range(nbuf):
            i = (n_groups - 1) * nbuf + b
            gat_dma(b).wait()
            out_dma(i, b).start()
        for b in range(nbuf):
            i = (n_groups - 1) * nbuf + b
            out_dma(i, b).wait()

    return gather_k


def kernel(token_ids, W):
    S, T = token_ids.shape
    V, D = W.shape
    B = S * T
    idx = token_ids.reshape(B).astype(jnp.int32)
    out = _make_gather(V, D, B, 512, 4)(W, idx)
    return out.reshape(S, T, D)
